# baseline (speedup 1.0000x reference)
import jax
import jax.numpy as jnp
from jax import lax
from jax.experimental import pallas as pl
from jax.experimental.pallas import tpu as pltpu
from jax.experimental.pallas import tpu_sc as plsc

_NTOKEN = 1000000
_NINP = 64
_SEQ = 50
_BATCH = 16384

_NC = 2
_NS = 16
_NW = _NC * _NS
_BPW = _BATCH // _NW
_JGRP = _BPW // 128
_CBLK = 32768
_NCH = 5
_SPC = _SEQ // _NCH


_NBLK = (_NTOKEN + _CBLK - 1) // _CBLK
_TPAD = _NBLK * _CBLK


def _matvec_body(tblT_ref, w_ref, o_hbm, scr, sem):
    i = pl.program_id(0)
    scr[...] = jnp.sum(tblT_ref[...] * w_ref[...], axis=0)
    cp = pltpu.make_async_copy(scr, o_hbm.at[pl.ds(i * _CBLK, _CBLK)], sem)
    cp.start()
    cp.wait()


def _matvec(tableT, w_col):
    return pl.pallas_call(
        _matvec_body,
        grid=(_NBLK,),
        in_specs=[
            pl.BlockSpec((_NINP, _CBLK), lambda i: (0, i)),
            pl.BlockSpec((_NINP, 1), lambda i: (0, 0)),
        ],
        out_specs=pl.BlockSpec(memory_space=pltpu.MemorySpace.HBM),
        out_shape=jax.ShapeDtypeStruct((_TPAD,), jnp.float32),
        scratch_shapes=[
            pltpu.VMEM((_CBLK,), jnp.float32),
            pltpu.SemaphoreType.DMA,
        ],
    )(tableT, w_col)


def _sc_body(t_hbm, idx_hbm, b_hbm, out_hbm, idx_v, vals_v, acc_v, b_v, *sems):
    wid = lax.axis_index("s") * _NC + lax.axis_index("c")
    pltpu.sync_copy(idx_hbm.at[wid], idx_v)
    pltpu.sync_copy(b_hbm, b_v)
    nv = _SPC * _BPW
    copies = []
    for c in range(_NCH):
        sl = pl.ds(c * nv, nv)
        copies.append(
            pltpu.async_copy(t_hbm.at[idx_v.at[sl]], vals_v.at[sl], sems[c]))
    zeros = jnp.zeros((16,), jnp.float32)
    for j in range(_BPW // 16):
        acc_v[pl.ds(j * 16, 16)] = zeros

    def _step(s, carry):
        base = s * _BPW
        for j in range(_BPW // 16):
            sl = pl.ds(j * 16, 16)
            acc_v[sl] += vals_v[pl.ds(base + j * 16, 16)]
        return carry

    for c in range(_NCH):
        copies[c].wait()
        lax.fori_loop(c * _SPC, (c + 1) * _SPC, _step, 0)

    bvec = b_v[...]
    inv = jnp.float32(1.0 / _SEQ)
    one = jnp.float32(1.0)
    for j in range(_BPW // 16):
        sl = pl.ds(j * 16, 16)
        x = acc_v[sl] * inv + bvec
        acc_v[sl] = one / (one + jnp.exp(-x))
    pltpu.sync_copy(acc_v, out_hbm.at[pl.ds(wid * _BPW, _BPW)])


def _sc_pool(t_flat, idx3, b16):
    mesh = plsc.VectorSubcoreMesh(core_axis_name="c", subcore_axis_name="s")
    f = pl.kernel(
        _sc_body,
        out_type=jax.ShapeDtypeStruct((_BATCH,), jnp.float32),
        mesh=mesh,
        scratch_types=[
            pltpu.VMEM((_SEQ * _BPW,), jnp.int32),
            pltpu.VMEM((_SEQ * _BPW,), jnp.float32),
            pltpu.VMEM((_BPW,), jnp.float32),
            pltpu.VMEM((16,), jnp.float32),
        ] + [pltpu.SemaphoreType.DMA] * _NCH,
    )
    return f(t_flat, idx3, b16)


def kernel(input, table, W, b):
    idx3 = (input.astype(jnp.int32)
            .reshape(_SEQ, _NW, _BPW)
            .transpose(1, 0, 2)
            .reshape(_NW, _SEQ * _BPW))
    w_col = W.reshape(_NINP, 1).astype(jnp.float32)
    t = _matvec(table.T, w_col)
    b16 = jnp.broadcast_to(b.astype(jnp.float32), (16,))
    out = _sc_pool(t, idx3, b16)
    return out.reshape(_BATCH, 1)


# speedup vs baseline: 1.0616x; 1.0616x over previous
import jax
import jax.numpy as jnp
from jax import lax
from jax.experimental import pallas as pl
from jax.experimental.pallas import tpu as pltpu
from jax.experimental.pallas import tpu_sc as plsc

_NTOKEN = 1000000
_NINP = 64
_SEQ = 50
_BATCH = 16384

_NC = 2
_NS = 16
_NW = _NC * _NS
_BPW = _BATCH // _NW
_JGRP = _BPW // 128
_CBLK = 32768
_NCH = 10
_SPC = _SEQ // _NCH


def _matvec_body(tblT_ref, w_ref, o_ref):
    prod = tblT_ref[...] * w_ref[...]
    o_ref[...] = jnp.sum(prod, axis=0)


def _matvec(tableT, w_col):
    return pl.pallas_call(
        _matvec_body,
        grid=((_NTOKEN + _CBLK - 1) // _CBLK,),
        in_specs=[
            pl.BlockSpec((_NINP, _CBLK), lambda i: (0, i)),
            pl.BlockSpec((_NINP, 1), lambda i: (0, 0)),
        ],
        out_specs=pl.BlockSpec((_CBLK,), lambda i: (i,)),
        out_shape=jax.ShapeDtypeStruct((_NTOKEN,), jnp.float32),
    )(tableT, w_col)


def _sc_body(t_hbm, idx_hbm, b_hbm, out_hbm, idx_v, vals_v, acc_v, b_v, *sems):
    wid = lax.axis_index("s") * _NC + lax.axis_index("c")
    nv = _SPC * _BPW
    pltpu.sync_copy(b_hbm, b_v)
    copies = []
    for c in range(_NCH):
        sl = pl.ds(c * nv, nv)
        pltpu.sync_copy(idx_hbm.at[wid, sl], idx_v.at[sl])
        copies.append(
            pltpu.async_copy(t_hbm.at[idx_v.at[sl]], vals_v.at[sl], sems[c]))
    zeros = jnp.zeros((16,), jnp.float32)
    for j in range(_BPW // 16):
        acc_v[pl.ds(j * 16, 16)] = zeros

    def _step(s, carry):
        base = s * _BPW
        for j in range(_BPW // 16):
            sl = pl.ds(j * 16, 16)
            acc_v[sl] += vals_v[pl.ds(base + j * 16, 16)]
        return carry

    for c in range(_NCH):
        copies[c].wait()
        lax.fori_loop(c * _SPC, (c + 1) * _SPC, _step, 0)

    bvec = b_v[...]
    inv = jnp.float32(1.0 / _SEQ)
    one = jnp.float32(1.0)
    for j in range(_BPW // 16):
        sl = pl.ds(j * 16, 16)
        x = acc_v[sl] * inv + bvec
        acc_v[sl] = one / (one + jnp.exp(-x))
    pltpu.sync_copy(acc_v, out_hbm.at[pl.ds(wid * _BPW, _BPW)])


def _sc_pool(t_flat, idx3, b16):
    mesh = plsc.VectorSubcoreMesh(core_axis_name="c", subcore_axis_name="s")
    f = pl.kernel(
        _sc_body,
        out_type=jax.ShapeDtypeStruct((_BATCH,), jnp.float32),
        mesh=mesh,
        scratch_types=[
            pltpu.VMEM((_SEQ * _BPW,), jnp.int32),
            pltpu.VMEM((_SEQ * _BPW,), jnp.float32),
            pltpu.VMEM((_BPW,), jnp.float32),
            pltpu.VMEM((16,), jnp.float32),
        ] + [pltpu.SemaphoreType.DMA] * _NCH,
    )
    return f(t_flat, idx3, b16)


def kernel(input, table, W, b):
    idx3 = (input.astype(jnp.int32)
            .reshape(_SEQ, _NW, _BPW)
            .transpose(1, 0, 2)
            .reshape(_NW, _SEQ * _BPW))
    w_col = W.reshape(_NINP, 1).astype(jnp.float32)
    t = _matvec(table.T, w_col)
    b16 = jnp.broadcast_to(b.astype(jnp.float32), (16,))
    out = _sc_pool(t, idx3, b16)
    return out.reshape(_BATCH, 1)
